# baseline (speedup 1.0000x reference)
import functools

import jax
import jax.numpy as jnp
from jax import lax
from jax.experimental import pallas as pl
from jax.experimental.pallas import tpu as pltpu
from jax.experimental.pallas import tpu_sc as plsc

N = 10000
E = 320000
IN_DIM = 128
HID = 512
LAT = 64
NEG_SLOPE = 0.2

NC = 2
NS = 16
LANES = 16
NW = NC * NS
K = 80

EPW = E // NW
CHW = EPW // K
EPS = E // NS
CHS = EPS // K
VEC = K // LANES
FA = 32
NBUF = 5

_HI = lax.Precision.HIGHEST
_SC_PARAMS = pltpu.CompilerParams(needs_layout_passes=False,
                                  use_tc_tiling_on_sc=False)


def _mesh():
    return plsc.VectorSubcoreMesh(
        core_axis_name="c", subcore_axis_name="s", num_cores=NC, num_subcores=NS
    )


def _attn_body(x_ref, w1_ref, att2_ref, out_ref):
    w12 = jnp.dot(w1_ref[...], att2_ref[...],
                  preferred_element_type=jnp.float32, precision=_HI)
    out_ref[...] = jnp.dot(x_ref[...], w12,
                           preferred_element_type=jnp.float32, precision=_HI)


def _attn_scalars(x, W1, att2):
    return pl.pallas_call(
        _attn_body,
        out_shape=jax.ShapeDtypeStruct((N, 2), jnp.float32),
    )(x, W1, att2)


BN = 1000


def _dense1_body(agg_a_ref, agg_b_ref, den_ref, w1_ref, w2_ref, wm_ref,
                 bm_ref, wv_ref, bv_ref, eps_ref, mean_ref, lv_ref, z_ref):
    p = jnp.concatenate([agg_a_ref[0], agg_a_ref[1],
                         agg_b_ref[0], agg_b_ref[1]], axis=1)
    den = jnp.sum(den_ref[...], axis=0)[:, None] + 1e-16
    aggn = p / den
    out1 = jnp.dot(aggn, w1_ref[...],
                   preferred_element_type=jnp.float32, precision=_HI)
    h1 = jnp.where(out1 > 0, out1, jnp.exp(jnp.minimum(out1, 0.0)) - 1.0)
    hidden = jnp.dot(h1, w2_ref[...],
                     preferred_element_type=jnp.float32, precision=_HI)
    dn = (((1,), (1,)), ((), ()))
    mean = lax.dot_general(hidden, wm_ref[...], dn,
                           preferred_element_type=jnp.float32,
                           precision=_HI) + bm_ref[...]
    lv = lax.dot_general(hidden, wv_ref[...], dn,
                         preferred_element_type=jnp.float32,
                         precision=_HI) + bv_ref[...]
    lv = jnp.clip(lv, -10.0, 10.0)
    std = jnp.sqrt(jnp.exp(0.5 * lv) + 1e-8)
    mean_ref[...] = mean
    lv_ref[...] = lv
    z_ref[...] = mean + eps_ref[...] * std


def _dense1(agg_a, agg_b, den_t, W1, W2, Wm, bm2, Wv, bv2, eps):
    grid = (N // BN,)
    full = lambda shape: pl.BlockSpec(shape, lambda i: tuple(0 for _ in shape))
    out = jax.ShapeDtypeStruct((N, LAT), jnp.float32)
    return pl.pallas_call(
        _dense1_body,
        grid=grid,
        in_specs=[
            pl.BlockSpec((NC, BN, FA), lambda i: (0, i, 0)),
            pl.BlockSpec((NC, BN, FA), lambda i: (0, i, 0)),
            pl.BlockSpec((NW, BN), lambda i: (i, 0)),
            full((IN_DIM, HID)),
            full((HID, LAT)),
            full((LAT, LAT)),
            full((1, LAT)),
            full((LAT, LAT)),
            full((1, LAT)),
            pl.BlockSpec((BN, LAT), lambda i: (i, 0)),
        ],
        out_specs=[pl.BlockSpec((BN, LAT), lambda i: (i, 0))] * 3,
        out_shape=[out, out, out],
    )(agg_a, agg_b, den_t, W1, W2, Wm, bm2, Wv, bv2, eps)


def _dense2_body(agg_ref, den_ref, w2_ref, w1_ref, mu_ref):
    p = jnp.concatenate([agg_ref[0], agg_ref[1]], axis=1)
    den = jnp.sum(den_ref[...], axis=0)[:, None] + 1e-16
    aggn = p / den
    dn = (((1,), (1,)), ((), ()))
    pre = lax.dot_general(aggn, w2_ref[...], dn,
                          preferred_element_type=jnp.float32, precision=_HI)
    h3 = jnp.where(pre > 0, pre, jnp.exp(jnp.minimum(pre, 0.0)) - 1.0)
    recon = lax.dot_general(h3, w1_ref[...], dn,
                            preferred_element_type=jnp.float32, precision=_HI)
    mu_ref[...] = jnp.maximum(recon, 0.0) + jnp.log(1.0 + jnp.exp(-jnp.abs(recon)))


def _dense2(agg_z, den_t, W2, W1):
    grid = (N // BN,)
    full = lambda shape: pl.BlockSpec(shape, lambda i: tuple(0 for _ in shape))
    return pl.pallas_call(
        _dense2_body,
        grid=grid,
        in_specs=[
            pl.BlockSpec((NC, BN, FA), lambda i: (0, i, 0)),
            pl.BlockSpec((NW, BN), lambda i: (i, 0)),
            full((HID, LAT)),
            full((IN_DIM, HID)),
        ],
        out_specs=[pl.BlockSpec((BN, IN_DIM), lambda i: (i, 0))],
        out_shape=[jax.ShapeDtypeStruct((N, IN_DIM), jnp.float32)],
    )(agg_z, den_t, W2, W1)[0]


def _sc_edge(a_src, a_dst, src_w, dst_w, zden):
    out_type = (
        jax.ShapeDtypeStruct((E,), jnp.float32),
        jax.ShapeDtypeStruct((NW, N), jnp.float32),
    )
    scratch = [
        pltpu.VMEM((N,), jnp.float32),
        pltpu.VMEM((N,), jnp.float32),
        pltpu.VMEM((N,), jnp.float32),
        pltpu.VMEM((CHW, K), jnp.int32),
        pltpu.VMEM((CHW, K), jnp.int32),
        pltpu.VMEM((EPW,), jnp.float32),
    ]

    @functools.partial(pl.kernel, out_type=out_type, mesh=_mesh(),
                       scratch_types=scratch, compiler_params=_SC_PARAMS)
    def k(a_src_hbm, a_dst_hbm, src_hbm, dst_hbm, zden_hbm,
          ex_hbm, den_out,
          a_src_v, a_dst_v, den_v, sidx, didx, ex_v):
        c = lax.axis_index("c")
        s = lax.axis_index("s")
        wid = s * NC + c
        pltpu.sync_copy(a_src_hbm, a_src_v)
        pltpu.sync_copy(a_dst_hbm, a_dst_v)
        pltpu.sync_copy(zden_hbm, den_v)
        pltpu.sync_copy(src_hbm.at[wid], sidx)
        pltpu.sync_copy(dst_hbm.at[wid], didx)

        @pl.loop(0, CHW)
        def _(j):
            for v in range(VEC):
                si = sidx[j, pl.ds(v * LANES, LANES)]
                di = didx[j, pl.ds(v * LANES, LANES)]
                e = plsc.load_gather(a_src_v, [si]) + plsc.load_gather(a_dst_v, [di])
                e = jnp.maximum(e, NEG_SLOPE * e)
                exv = jnp.exp(e)
                ex_v[pl.ds(j * K + v * LANES, LANES)] = exv
                plsc.addupdate_scatter(den_v, [di], exv)

        pltpu.sync_copy(ex_v, ex_hbm.at[pl.ds(wid * EPW, EPW)])
        pltpu.sync_copy(den_v, den_out.at[wid])

    return k(a_src, a_dst, src_w, dst_w, zden)


def _sc_agg(table, ex, src_r, dst_r, zag):
    out_type = jax.ShapeDtypeStruct((NC, N, FA), jnp.float32)
    scratch = (
        [pltpu.VMEM((CHS, K), jnp.int32),
         pltpu.VMEM((CHS, K), jnp.int32),
         pltpu.VMEM((EPS,), jnp.float32)]
        + [pltpu.VMEM((K, FA), jnp.float32) for _ in range(NBUF)]
        + [pltpu.SemaphoreType.DMA for _ in range(NBUF + 1)]
        + [pltpu.VMEM_SHARED((N, FA), jnp.float32)]
    )
    rows_per_sub = N // NS

    @functools.partial(pl.kernel, out_type=out_type, mesh=_mesh(),
                       scratch_types=scratch, compiler_params=_SC_PARAMS)
    def k(t_hbm, ex_hbm, src_hbm, dst_hbm, zag_hbm, agg_out,
          sidx, didx, ex_v, r0, r1, r2, r3, r4, g0, g1, g2, g3, g4, ssem,
          agg_sp):
        rows = (r0, r1, r2, r3, r4)
        gsem = (g0, g1, g2, g3, g4)
        c = lax.axis_index("c")
        s = lax.axis_index("s")
        pltpu.sync_copy(src_hbm.at[s], sidx)
        pltpu.sync_copy(dst_hbm.at[s], didx)
        pltpu.sync_copy(ex_hbm.at[pl.ds(s * EPS, EPS)], ex_v)
        rsl = pl.ds(s * rows_per_sub, rows_per_sub)
        pltpu.sync_copy(zag_hbm.at[rsl], agg_sp.at[rsl])

        coff = c * N

        @pl.loop(0, CHS)
        def _(j):
            for v in range(VEC):
                si = sidx[j, pl.ds(v * LANES, LANES)]
                sidx[j, pl.ds(v * LANES, LANES)] = si + coff

        plsc.subcore_barrier()

        def mult(j, buf):
            for v in range(VEC):
                for l in range(LANES):
                    bvec = plsc.load_gather(
                        ex_v, [jnp.full((LANES,), j * K + v * LANES + l, jnp.int32)])
                    r = v * LANES + l
                    for f in range(FA // LANES):
                        sl = (r, pl.ds(f * LANES, LANES))
                        buf[sl] = buf[sl] * bvec

        @pl.loop(0, CHS // NBUF)
        def _(h):
            j0 = NBUF * h
            gds = [pltpu.async_copy(t_hbm.at[sidx.at[j0 + b]], rows[b], gsem[b])
                   for b in range(NBUF)]
            sds = []
            for b in range(NBUF):
                gds[b].wait()
                mult(j0 + b, rows[b])
                sds.append(pltpu.async_copy(
                    rows[b], agg_sp.at[didx.at[j0 + b]], ssem, add=True))
            for d in sds:
                d.wait()

        plsc.subcore_barrier()
        pltpu.sync_copy(agg_sp.at[rsl], agg_out.at[c, rsl])

    return k(table, ex, src_r, dst_r, zag)


def kernel(x, edge_index, W1, att_src1, att_dst1, W2, Wm, bm, Wv, bv, log_theta):
    src = edge_index[0]
    dst = edge_index[1]
    src_w = src.reshape(NW, CHW, K)
    dst_w = dst.reshape(NW, CHW, K)
    src_r = src.reshape(NS, CHS, K)
    dst_r = dst.reshape(NS, CHS, K)
    att2 = jnp.stack([att_src1, att_dst1], axis=1)

    a2 = _attn_scalars(x, W1, att2)
    a_src = a2[:, 0]
    a_dst = a2[:, 1]

    zden = jnp.zeros((N,), jnp.float32)
    zag = jnp.zeros((N, FA), jnp.float32)

    ex, den = _sc_edge(a_src, a_dst, src_w, dst_w, zden)
    den_t = den.reshape(NW, N // BN, BN).transpose(1, 0, 2)
    den_t = den_t.reshape(N // BN * NW, BN)

    x_a = jnp.concatenate([x[:, 0 * FA:1 * FA], x[:, 1 * FA:2 * FA]], axis=0)
    x_b = jnp.concatenate([x[:, 2 * FA:3 * FA], x[:, 3 * FA:4 * FA]], axis=0)
    agg_a = _sc_agg(x_a, ex, src_r, dst_r, zag)
    agg_b = _sc_agg(x_b, ex, src_r, dst_r, zag)

    eps = jax.random.normal(jax.random.key(42), (N, LAT), jnp.float32)
    mean, log_var, z = _dense1(agg_a, agg_b, den_t, W1, W2, Wm,
                               bm.reshape(1, LAT), Wv, bv.reshape(1, LAT), eps)

    zflat = jnp.concatenate([z[:, :FA], z[:, FA:]], axis=0)
    agg_z = _sc_agg(zflat, ex, src_r, dst_r, zag)
    mu = _dense2(agg_z, den_t, W2, W1)
    theta = jnp.exp(log_theta)
    return (mean, log_var, mu, theta, z)


# speedup vs baseline: 1.2793x; 1.2793x over previous
import functools

import jax
import jax.numpy as jnp
from jax import lax
from jax.experimental import pallas as pl
from jax.experimental.pallas import tpu as pltpu
from jax.experimental.pallas import tpu_sc as plsc

N = 10000
E = 320000
IN_DIM = 128
HID = 512
LAT = 64
NEG_SLOPE = 0.2

NC = 2
NS = 16
LANES = 16
NW = NC * NS
K = 80

EPW = E // NW
CHW = EPW // K
EPS = E // NS
CHS = EPS // K
VEC = K // LANES
FA = 32
NBUF = 2

_HI = lax.Precision.HIGHEST
_SC_PARAMS = pltpu.CompilerParams(needs_layout_passes=False,
                                  use_tc_tiling_on_sc=False)


def _mesh():
    return plsc.VectorSubcoreMesh(
        core_axis_name="c", subcore_axis_name="s", num_cores=NC, num_subcores=NS
    )


def _attn_body(x_ref, w1_ref, att2_ref, out_ref):
    w12 = jnp.dot(w1_ref[...], att2_ref[...],
                  preferred_element_type=jnp.float32, precision=_HI)
    out_ref[...] = jnp.dot(x_ref[...], w12,
                           preferred_element_type=jnp.float32, precision=_HI)


def _attn_scalars(x, W1, att2):
    return pl.pallas_call(
        _attn_body,
        out_shape=jax.ShapeDtypeStruct((N, 2), jnp.float32),
    )(x, W1, att2)


BN = 1000


def _dense1_body(agg_a_ref, agg_b_ref, den_ref, w1_ref, w2_ref, wm_ref,
                 bm_ref, wv_ref, bv_ref, eps_ref, mean_ref, lv_ref, z_ref):
    p = jnp.concatenate([agg_a_ref[0], agg_a_ref[1],
                         agg_b_ref[0], agg_b_ref[1]], axis=1)
    den = jnp.sum(den_ref[...], axis=0)[:, None] + 1e-16
    aggn = p / den
    out1 = jnp.dot(aggn, w1_ref[...],
                   preferred_element_type=jnp.float32, precision=_HI)
    h1 = jnp.where(out1 > 0, out1, jnp.exp(jnp.minimum(out1, 0.0)) - 1.0)
    hidden = jnp.dot(h1, w2_ref[...],
                     preferred_element_type=jnp.float32, precision=_HI)
    dn = (((1,), (1,)), ((), ()))
    mean = lax.dot_general(hidden, wm_ref[...], dn,
                           preferred_element_type=jnp.float32,
                           precision=_HI) + bm_ref[...]
    lv = lax.dot_general(hidden, wv_ref[...], dn,
                         preferred_element_type=jnp.float32,
                         precision=_HI) + bv_ref[...]
    lv = jnp.clip(lv, -10.0, 10.0)
    std = jnp.sqrt(jnp.exp(0.5 * lv) + 1e-8)
    mean_ref[...] = mean
    lv_ref[...] = lv
    z_ref[...] = mean + eps_ref[...] * std


def _dense1(agg_a, agg_b, den_t, W1, W2, Wm, bm2, Wv, bv2, eps):
    grid = (N // BN,)
    full = lambda shape: pl.BlockSpec(shape, lambda i: tuple(0 for _ in shape))
    out = jax.ShapeDtypeStruct((N, LAT), jnp.float32)
    return pl.pallas_call(
        _dense1_body,
        grid=grid,
        in_specs=[
            pl.BlockSpec((NC, BN, FA), lambda i: (0, i, 0)),
            pl.BlockSpec((NC, BN, FA), lambda i: (0, i, 0)),
            pl.BlockSpec((NW, BN), lambda i: (i, 0)),
            full((IN_DIM, HID)),
            full((HID, LAT)),
            full((LAT, LAT)),
            full((1, LAT)),
            full((LAT, LAT)),
            full((1, LAT)),
            pl.BlockSpec((BN, LAT), lambda i: (i, 0)),
        ],
        out_specs=[pl.BlockSpec((BN, LAT), lambda i: (i, 0))] * 3,
        out_shape=[out, out, out],
    )(agg_a, agg_b, den_t, W1, W2, Wm, bm2, Wv, bv2, eps)


def _dense2_body(agg_ref, den_ref, w2_ref, w1_ref, mu_ref):
    p = jnp.concatenate([agg_ref[0], agg_ref[1]], axis=1)
    den = jnp.sum(den_ref[...], axis=0)[:, None] + 1e-16
    aggn = p / den
    dn = (((1,), (1,)), ((), ()))
    pre = lax.dot_general(aggn, w2_ref[...], dn,
                          preferred_element_type=jnp.float32, precision=_HI)
    h3 = jnp.where(pre > 0, pre, jnp.exp(jnp.minimum(pre, 0.0)) - 1.0)
    recon = lax.dot_general(h3, w1_ref[...], dn,
                            preferred_element_type=jnp.float32, precision=_HI)
    mu_ref[...] = jnp.maximum(recon, 0.0) + jnp.log(1.0 + jnp.exp(-jnp.abs(recon)))


def _dense2(agg_z, den_t, W2, W1):
    grid = (N // BN,)
    full = lambda shape: pl.BlockSpec(shape, lambda i: tuple(0 for _ in shape))
    return pl.pallas_call(
        _dense2_body,
        grid=grid,
        in_specs=[
            pl.BlockSpec((NC, BN, FA), lambda i: (0, i, 0)),
            pl.BlockSpec((NW, BN), lambda i: (i, 0)),
            full((HID, LAT)),
            full((IN_DIM, HID)),
        ],
        out_specs=[pl.BlockSpec((BN, IN_DIM), lambda i: (i, 0))],
        out_shape=[jax.ShapeDtypeStruct((N, IN_DIM), jnp.float32)],
    )(agg_z, den_t, W2, W1)[0]


def _sc_edge(a_src, a_dst, src_w, dst_w, zden):
    out_type = (
        jax.ShapeDtypeStruct((E,), jnp.float32),
        jax.ShapeDtypeStruct((NW, N), jnp.float32),
    )
    scratch = [
        pltpu.VMEM((N,), jnp.float32),
        pltpu.VMEM((N,), jnp.float32),
        pltpu.VMEM((N,), jnp.float32),
        pltpu.VMEM((CHW, K), jnp.int32),
        pltpu.VMEM((CHW, K), jnp.int32),
        pltpu.VMEM((EPW,), jnp.float32),
    ]

    @functools.partial(pl.kernel, out_type=out_type, mesh=_mesh(),
                       scratch_types=scratch, compiler_params=_SC_PARAMS)
    def k(a_src_hbm, a_dst_hbm, src_hbm, dst_hbm, zden_hbm,
          ex_hbm, den_out,
          a_src_v, a_dst_v, den_v, sidx, didx, ex_v):
        c = lax.axis_index("c")
        s = lax.axis_index("s")
        wid = s * NC + c
        pltpu.sync_copy(a_src_hbm, a_src_v)
        pltpu.sync_copy(a_dst_hbm, a_dst_v)
        pltpu.sync_copy(zden_hbm, den_v)
        pltpu.sync_copy(src_hbm.at[wid], sidx)
        pltpu.sync_copy(dst_hbm.at[wid], didx)

        @pl.loop(0, CHW)
        def _(j):
            for v in range(VEC):
                si = sidx[j, pl.ds(v * LANES, LANES)]
                di = didx[j, pl.ds(v * LANES, LANES)]
                e = plsc.load_gather(a_src_v, [si]) + plsc.load_gather(a_dst_v, [di])
                e = jnp.maximum(e, NEG_SLOPE * e)
                exv = jnp.exp(e)
                ex_v[pl.ds(j * K + v * LANES, LANES)] = exv
                plsc.addupdate_scatter(den_v, [di], exv)

        pltpu.sync_copy(ex_v, ex_hbm.at[pl.ds(wid * EPW, EPW)])
        pltpu.sync_copy(den_v, den_out.at[wid])

    return k(a_src, a_dst, src_w, dst_w, zden)


def _sc_agg(table, ex, src_r, dst_r, zag):
    out_type = jax.ShapeDtypeStruct((NC, N, FA), jnp.float32)
    scratch = (
        [pltpu.VMEM((CHS, K), jnp.int32),
         pltpu.VMEM((CHS, K), jnp.int32),
         pltpu.VMEM((EPS,), jnp.float32)]
        + [pltpu.VMEM((K, FA), jnp.float32) for _ in range(NBUF)]
        + [pltpu.SemaphoreType.DMA for _ in range(NBUF + 1)]
        + [pltpu.VMEM_SHARED((N, FA), jnp.float32)]
    )
    rows_per_sub = N // NS

    @functools.partial(pl.kernel, out_type=out_type, mesh=_mesh(),
                       scratch_types=scratch, compiler_params=_SC_PARAMS)
    def k(t_hbm, ex_hbm, src_hbm, dst_hbm, zag_hbm, agg_out,
          sidx, didx, ex_v, r0, r1, g0, g1, ssem,
          agg_sp):
        rows = (r0, r1)
        gsem = (g0, g1)
        c = lax.axis_index("c")
        s = lax.axis_index("s")
        pltpu.sync_copy(src_hbm.at[s], sidx)
        pltpu.sync_copy(dst_hbm.at[s], didx)
        pltpu.sync_copy(ex_hbm.at[pl.ds(s * EPS, EPS)], ex_v)
        rsl = pl.ds(s * rows_per_sub, rows_per_sub)
        pltpu.sync_copy(zag_hbm.at[rsl], agg_sp.at[rsl])

        coff = c * N

        @pl.loop(0, CHS)
        def _(j):
            for v in range(VEC):
                si = sidx[j, pl.ds(v * LANES, LANES)]
                sidx[j, pl.ds(v * LANES, LANES)] = si + coff

        plsc.subcore_barrier()

        def mult(j, buf):
            for v in range(VEC):
                for l in range(LANES):
                    bvec = plsc.load_gather(
                        ex_v, [jnp.full((LANES,), j * K + v * LANES + l, jnp.int32)])
                    r = v * LANES + l
                    for f in range(FA // LANES):
                        sl = (r, pl.ds(f * LANES, LANES))
                        buf[sl] = buf[sl] * bvec

        @pl.loop(0, CHS // NBUF)
        def _(h):
            j0 = NBUF * h
            gds = [pltpu.async_copy(t_hbm.at[sidx.at[j0 + b]], rows[b], gsem[b])
                   for b in range(NBUF)]
            sds = []
            for b in range(NBUF):
                gds[b].wait()
                mult(j0 + b, rows[b])
                sds.append(pltpu.async_copy(
                    rows[b], agg_sp.at[didx.at[j0 + b]], ssem, add=True))
            for d in sds:
                d.wait()

        plsc.subcore_barrier()
        pltpu.sync_copy(agg_sp.at[rsl], agg_out.at[c, rsl])

    return k(table, ex, src_r, dst_r, zag)


def kernel(x, edge_index, W1, att_src1, att_dst1, W2, Wm, bm, Wv, bv, log_theta):
    src = edge_index[0]
    dst = edge_index[1]
    src_w = src.reshape(NW, CHW, K)
    dst_w = dst.reshape(NW, CHW, K)
    src_r = src.reshape(NS, CHS, K)
    dst_r = dst.reshape(NS, CHS, K)
    att2 = jnp.stack([att_src1, att_dst1], axis=1)

    a2 = _attn_scalars(x, W1, att2)
    a_src = a2[:, 0]
    a_dst = a2[:, 1]

    zden = jnp.zeros((N,), jnp.float32)
    zag = jnp.zeros((N, FA), jnp.float32)

    ex, den = _sc_edge(a_src, a_dst, src_w, dst_w, zden)
    den_t = den.reshape(NW, N // BN, BN).transpose(1, 0, 2)
    den_t = den_t.reshape(N // BN * NW, BN)

    x_a = jnp.concatenate([x[:, 0 * FA:1 * FA], x[:, 1 * FA:2 * FA]], axis=0)
    x_b = jnp.concatenate([x[:, 2 * FA:3 * FA], x[:, 3 * FA:4 * FA]], axis=0)
    agg_a = _sc_agg(x_a, ex, src_r, dst_r, zag)
    agg_b = _sc_agg(x_b, ex, src_r, dst_r, zag)

    eps = jax.random.normal(jax.random.key(42), (N, LAT), jnp.float32)
    mean, log_var, z = _dense1(agg_a, agg_b, den_t, W1, W2, Wm,
                               bm.reshape(1, LAT), Wv, bv.reshape(1, LAT), eps)

    zflat = jnp.concatenate([z[:, :FA], z[:, FA:]], axis=0)
    agg_z = _sc_agg(zflat, ex, src_r, dst_r, zag)
    mu = _dense2(agg_z, den_t, W2, W1)
    theta = jnp.exp(log_theta)
    return (mean, log_var, mu, theta, z)


# speedup vs baseline: 1.2948x; 1.0122x over previous
import functools

import jax
import jax.numpy as jnp
from jax import lax
from jax.experimental import pallas as pl
from jax.experimental.pallas import tpu as pltpu
from jax.experimental.pallas import tpu_sc as plsc

N = 10000
E = 320000
IN_DIM = 128
HID = 512
LAT = 64
NEG_SLOPE = 0.2

NC = 2
NS = 16
LANES = 16
NW = NC * NS
K = 80

EPW = E // NW
CHW = EPW // K
EPS = E // NS
CHS = EPS // K
VEC = K // LANES
FA = 32
NBUF = 2

_HI = lax.Precision.HIGHEST
_SC_PARAMS = pltpu.CompilerParams(needs_layout_passes=False,
                                  use_tc_tiling_on_sc=False)


def _mesh():
    return plsc.VectorSubcoreMesh(
        core_axis_name="c", subcore_axis_name="s", num_cores=NC, num_subcores=NS
    )


def _attn_body(x_ref, w1_ref, att2_ref, out_ref, xq_ref):
    w12 = jnp.dot(w1_ref[...], att2_ref[...],
                  preferred_element_type=jnp.float32, precision=_HI)
    x = x_ref[...]
    out_ref[...] = jnp.dot(x, w12,
                           preferred_element_type=jnp.float32, precision=_HI)
    xq_ref[...] = jnp.stack([x[:, 0 * FA:1 * FA], x[:, 1 * FA:2 * FA],
                             x[:, 2 * FA:3 * FA], x[:, 3 * FA:4 * FA]])


def _attn_scalars(x, W1, att2):
    return pl.pallas_call(
        _attn_body,
        out_shape=[jax.ShapeDtypeStruct((N, 2), jnp.float32),
                   jax.ShapeDtypeStruct((4, N, FA), jnp.float32)],
    )(x, W1, att2)


BN = 1000


def _dense1_body(agg_a_ref, agg_b_ref, den_ref, w1_ref, w2_ref, wm_ref,
                 bm_ref, wv_ref, bv_ref, eps_ref, mean_ref, lv_ref, z_ref,
                 zq_ref):
    p = jnp.concatenate([agg_a_ref[0], agg_a_ref[1],
                         agg_b_ref[0], agg_b_ref[1]], axis=1)
    den = jnp.sum(den_ref[...], axis=0)[:, None] + 1e-16
    aggn = p / den
    out1 = jnp.dot(aggn, w1_ref[...],
                   preferred_element_type=jnp.float32, precision=_HI)
    h1 = jnp.where(out1 > 0, out1, jnp.exp(jnp.minimum(out1, 0.0)) - 1.0)
    hidden = jnp.dot(h1, w2_ref[...],
                     preferred_element_type=jnp.float32, precision=_HI)
    dn = (((1,), (1,)), ((), ()))
    mean = lax.dot_general(hidden, wm_ref[...], dn,
                           preferred_element_type=jnp.float32,
                           precision=_HI) + bm_ref[...]
    lv = lax.dot_general(hidden, wv_ref[...], dn,
                         preferred_element_type=jnp.float32,
                         precision=_HI) + bv_ref[...]
    lv = jnp.clip(lv, -10.0, 10.0)
    std = jnp.sqrt(jnp.exp(0.5 * lv) + 1e-8)
    mean_ref[...] = mean
    lv_ref[...] = lv
    z = mean + eps_ref[...] * std
    z_ref[...] = z
    zq_ref[...] = jnp.stack([z[:, :FA], z[:, FA:]])


def _dense1(agg_a, agg_b, den_t, W1, W2, Wm, bm2, Wv, bv2, eps):
    grid = (N // BN,)
    full = lambda shape: pl.BlockSpec(shape, lambda i: tuple(0 for _ in shape))
    out = jax.ShapeDtypeStruct((N, LAT), jnp.float32)
    return pl.pallas_call(
        _dense1_body,
        grid=grid,
        in_specs=[
            pl.BlockSpec((NC, BN, FA), lambda i: (0, i, 0)),
            pl.BlockSpec((NC, BN, FA), lambda i: (0, i, 0)),
            pl.BlockSpec((NW, BN), lambda i: (i, 0)),
            full((IN_DIM, HID)),
            full((HID, LAT)),
            full((LAT, LAT)),
            full((1, LAT)),
            full((LAT, LAT)),
            full((1, LAT)),
            pl.BlockSpec((BN, LAT), lambda i: (i, 0)),
        ],
        out_specs=[pl.BlockSpec((BN, LAT), lambda i: (i, 0))] * 3
        + [pl.BlockSpec((2, BN, FA), lambda i: (0, i, 0))],
        out_shape=[out, out, out,
                   jax.ShapeDtypeStruct((2, N, FA), jnp.float32)],
    )(agg_a, agg_b, den_t, W1, W2, Wm, bm2, Wv, bv2, eps)


def _dense2_body(agg_ref, den_ref, w2_ref, w1_ref, mu_ref):
    p = jnp.concatenate([agg_ref[0], agg_ref[1]], axis=1)
    den = jnp.sum(den_ref[...], axis=0)[:, None] + 1e-16
    aggn = p / den
    dn = (((1,), (1,)), ((), ()))
    pre = lax.dot_general(aggn, w2_ref[...], dn,
                          preferred_element_type=jnp.float32, precision=_HI)
    h3 = jnp.where(pre > 0, pre, jnp.exp(jnp.minimum(pre, 0.0)) - 1.0)
    recon = lax.dot_general(h3, w1_ref[...], dn,
                            preferred_element_type=jnp.float32, precision=_HI)
    mu_ref[...] = jnp.maximum(recon, 0.0) + jnp.log(1.0 + jnp.exp(-jnp.abs(recon)))


def _dense2(agg_z, den_t, W2, W1):
    grid = (N // BN,)
    full = lambda shape: pl.BlockSpec(shape, lambda i: tuple(0 for _ in shape))
    return pl.pallas_call(
        _dense2_body,
        grid=grid,
        in_specs=[
            pl.BlockSpec((NC, BN, FA), lambda i: (0, i, 0)),
            pl.BlockSpec((NW, BN), lambda i: (i, 0)),
            full((HID, LAT)),
            full((IN_DIM, HID)),
        ],
        out_specs=[pl.BlockSpec((BN, IN_DIM), lambda i: (i, 0))],
        out_shape=[jax.ShapeDtypeStruct((N, IN_DIM), jnp.float32)],
    )(agg_z, den_t, W2, W1)[0]


def _sc_edge(a_src, a_dst, src_w, dst_w, zden):
    out_type = (
        jax.ShapeDtypeStruct((E,), jnp.float32),
        jax.ShapeDtypeStruct((N // BN, NW, BN), jnp.float32),
    )
    scratch = [
        pltpu.VMEM((N,), jnp.float32),
        pltpu.VMEM((N,), jnp.float32),
        pltpu.VMEM((N,), jnp.float32),
        pltpu.VMEM((CHW, K), jnp.int32),
        pltpu.VMEM((CHW, K), jnp.int32),
        pltpu.VMEM((EPW,), jnp.float32),
    ]

    @functools.partial(pl.kernel, out_type=out_type, mesh=_mesh(),
                       scratch_types=scratch, compiler_params=_SC_PARAMS)
    def k(a_src_hbm, a_dst_hbm, src_hbm, dst_hbm, zden_hbm,
          ex_hbm, den_out,
          a_src_v, a_dst_v, den_v, sidx, didx, ex_v):
        c = lax.axis_index("c")
        s = lax.axis_index("s")
        wid = s * NC + c
        pltpu.sync_copy(a_src_hbm, a_src_v)
        pltpu.sync_copy(a_dst_hbm, a_dst_v)
        pltpu.sync_copy(zden_hbm, den_v)
        pltpu.sync_copy(src_hbm.at[wid], sidx)
        pltpu.sync_copy(dst_hbm.at[wid], didx)

        @pl.loop(0, CHW)
        def _(j):
            for v in range(VEC):
                si = sidx[j, pl.ds(v * LANES, LANES)]
                di = didx[j, pl.ds(v * LANES, LANES)]
                e = plsc.load_gather(a_src_v, [si]) + plsc.load_gather(a_dst_v, [di])
                e = jnp.maximum(e, NEG_SLOPE * e)
                exv = jnp.exp(e)
                ex_v[pl.ds(j * K + v * LANES, LANES)] = exv
                plsc.addupdate_scatter(den_v, [di], exv)

        pltpu.sync_copy(ex_v, ex_hbm.at[pl.ds(wid * EPW, EPW)])
        for bi in range(N // BN):
            pltpu.sync_copy(den_v.at[pl.ds(bi * BN, BN)], den_out.at[bi, wid])

    return k(a_src, a_dst, src_w, dst_w, zden)


def _sc_agg(table, ex, src_r, dst_r, zag):
    out_type = jax.ShapeDtypeStruct((NC, N, FA), jnp.float32)
    scratch = (
        [pltpu.VMEM((CHS, K), jnp.int32),
         pltpu.VMEM((CHS, K), jnp.int32),
         pltpu.VMEM((EPS,), jnp.float32)]
        + [pltpu.VMEM((K, FA), jnp.float32) for _ in range(NBUF)]
        + [pltpu.SemaphoreType.DMA for _ in range(NBUF + 1)]
        + [pltpu.VMEM_SHARED((N, FA), jnp.float32)]
    )
    rows_per_sub = N // NS

    @functools.partial(pl.kernel, out_type=out_type, mesh=_mesh(),
                       scratch_types=scratch, compiler_params=_SC_PARAMS)
    def k(t_hbm, ex_hbm, src_hbm, dst_hbm, zag_hbm, agg_out,
          sidx, didx, ex_v, r0, r1, g0, g1, ssem,
          agg_sp):
        rows = (r0, r1)
        gsem = (g0, g1)
        c = lax.axis_index("c")
        s = lax.axis_index("s")
        pltpu.sync_copy(src_hbm.at[s], sidx)
        pltpu.sync_copy(dst_hbm.at[s], didx)
        pltpu.sync_copy(ex_hbm.at[pl.ds(s * EPS, EPS)], ex_v)
        rsl = pl.ds(s * rows_per_sub, rows_per_sub)
        pltpu.sync_copy(zag_hbm.at[rsl], agg_sp.at[rsl])

        coff = c * N

        @pl.loop(0, CHS)
        def _(j):
            for v in range(VEC):
                si = sidx[j, pl.ds(v * LANES, LANES)]
                sidx[j, pl.ds(v * LANES, LANES)] = si + coff

        plsc.subcore_barrier()

        def mult(j, buf):
            for v in range(VEC):
                for l in range(LANES):
                    bvec = plsc.load_gather(
                        ex_v, [jnp.full((LANES,), j * K + v * LANES + l, jnp.int32)])
                    r = v * LANES + l
                    for f in range(FA // LANES):
                        sl = (r, pl.ds(f * LANES, LANES))
                        buf[sl] = buf[sl] * bvec

        @pl.loop(0, CHS // NBUF)
        def _(h):
            j0 = NBUF * h
            gds = [pltpu.async_copy(t_hbm.at[sidx.at[j0 + b]], rows[b], gsem[b])
                   for b in range(NBUF)]
            sds = []
            for b in range(NBUF):
                gds[b].wait()
                mult(j0 + b, rows[b])
                sds.append(pltpu.async_copy(
                    rows[b], agg_sp.at[didx.at[j0 + b]], ssem, add=True))
            for d in sds:
                d.wait()

        plsc.subcore_barrier()
        pltpu.sync_copy(agg_sp.at[rsl], agg_out.at[c, rsl])

    return k(table, ex, src_r, dst_r, zag)


def kernel(x, edge_index, W1, att_src1, att_dst1, W2, Wm, bm, Wv, bv, log_theta):
    src = edge_index[0]
    dst = edge_index[1]
    src_w = src.reshape(NW, CHW, K)
    dst_w = dst.reshape(NW, CHW, K)
    src_r = src.reshape(NS, CHS, K)
    dst_r = dst.reshape(NS, CHS, K)
    att2 = jnp.stack([att_src1, att_dst1], axis=1)

    a2, xq = _attn_scalars(x, W1, att2)
    a_src = a2[:, 0]
    a_dst = a2[:, 1]

    zden = jnp.zeros((N,), jnp.float32)
    zag = jnp.zeros((N, FA), jnp.float32)

    ex, den = _sc_edge(a_src, a_dst, src_w, dst_w, zden)
    den_t = den.reshape(N // BN * NW, BN)

    agg_a = _sc_agg(xq[0:2].reshape(2 * N, FA), ex, src_r, dst_r, zag)
    agg_b = _sc_agg(xq[2:4].reshape(2 * N, FA), ex, src_r, dst_r, zag)

    eps = jax.random.normal(jax.random.key(42), (N, LAT), jnp.float32)
    mean, log_var, z, zq = _dense1(agg_a, agg_b, den_t, W1, W2, Wm,
                                   bm.reshape(1, LAT), Wv,
                                   bv.reshape(1, LAT), eps)

    agg_z = _sc_agg(zq.reshape(2 * N, FA), ex, src_r, dst_r, zag)
    mu = _dense2(agg_z, den_t, W2, W1)
    theta = jnp.exp(log_theta)
    return (mean, log_var, mu, theta, z)


# speedup vs baseline: 1.3174x; 1.0174x over previous
import functools

import jax
import jax.numpy as jnp
from jax import lax
from jax.experimental import pallas as pl
from jax.experimental.pallas import tpu as pltpu
from jax.experimental.pallas import tpu_sc as plsc

N = 10000
E = 320000
IN_DIM = 128
HID = 512
LAT = 64
NEG_SLOPE = 0.2

NC = 2
NS = 16
LANES = 16
K = 80

EPS = E // NS
CHS = EPS // K
VEC = K // LANES
FA = 32
NBUF = 2
ZROWS = 125

_HI = lax.Precision.HIGHEST
_SC_PARAMS = pltpu.CompilerParams(needs_layout_passes=False,
                                  use_tc_tiling_on_sc=False)


def _mesh():
    return plsc.VectorSubcoreMesh(
        core_axis_name="c", subcore_axis_name="s", num_cores=NC, num_subcores=NS
    )


def _attn_body(x_ref, w1_ref, att2_ref, out_ref, xq_ref):
    w12 = jnp.dot(w1_ref[...], att2_ref[...],
                  preferred_element_type=jnp.float32, precision=_HI)
    x = x_ref[...]
    out_ref[...] = jnp.dot(x, w12,
                           preferred_element_type=jnp.float32, precision=_HI)
    xq_ref[...] = jnp.stack([x[:, 0 * FA:1 * FA], x[:, 1 * FA:2 * FA],
                             x[:, 2 * FA:3 * FA], x[:, 3 * FA:4 * FA]])


def _attn_scalars(x, W1, att2):
    return pl.pallas_call(
        _attn_body,
        out_shape=[jax.ShapeDtypeStruct((N, 2), jnp.float32),
                   jax.ShapeDtypeStruct((4, N, FA), jnp.float32)],
    )(x, W1, att2)


BN = 1000


def _dense1_body(agg_a_ref, agg_b_ref, den_ref, w1_ref, w2_ref, wm_ref,
                 bm_ref, wv_ref, bv_ref, eps_ref, mean_ref, lv_ref, z_ref,
                 zq_ref):
    p = jnp.concatenate([agg_a_ref[0], agg_a_ref[1],
                         agg_b_ref[0], agg_b_ref[1]], axis=1)
    den = jnp.sum(den_ref[...], axis=0)[:, None] + 1e-16
    aggn = p / den
    out1 = jnp.dot(aggn, w1_ref[...],
                   preferred_element_type=jnp.float32, precision=_HI)
    h1 = jnp.where(out1 > 0, out1, jnp.exp(jnp.minimum(out1, 0.0)) - 1.0)
    hidden = jnp.dot(h1, w2_ref[...],
                     preferred_element_type=jnp.float32, precision=_HI)
    dn = (((1,), (1,)), ((), ()))
    mean = lax.dot_general(hidden, wm_ref[...], dn,
                           preferred_element_type=jnp.float32,
                           precision=_HI) + bm_ref[...]
    lv = lax.dot_general(hidden, wv_ref[...], dn,
                         preferred_element_type=jnp.float32,
                         precision=_HI) + bv_ref[...]
    lv = jnp.clip(lv, -10.0, 10.0)
    std = jnp.sqrt(jnp.exp(0.5 * lv) + 1e-8)
    mean_ref[...] = mean
    lv_ref[...] = lv
    z = mean + eps_ref[...] * std
    z_ref[...] = z
    zq_ref[...] = jnp.stack([z[:, :FA], z[:, FA:]])


def _dense1(agg_a, agg_b, den_t, W1, W2, Wm, bm2, Wv, bv2, eps):
    grid = (N // BN,)
    full = lambda shape: pl.BlockSpec(shape, lambda i: tuple(0 for _ in shape))
    out = jax.ShapeDtypeStruct((N, LAT), jnp.float32)
    return pl.pallas_call(
        _dense1_body,
        grid=grid,
        in_specs=[
            pl.BlockSpec((NC, BN, FA), lambda i: (0, i, 0)),
            pl.BlockSpec((NC, BN, FA), lambda i: (0, i, 0)),
            pl.BlockSpec((NS, BN), lambda i: (i, 0)),
            full((IN_DIM, HID)),
            full((HID, LAT)),
            full((LAT, LAT)),
            full((1, LAT)),
            full((LAT, LAT)),
            full((1, LAT)),
            pl.BlockSpec((BN, LAT), lambda i: (i, 0)),
        ],
        out_specs=[pl.BlockSpec((BN, LAT), lambda i: (i, 0))] * 3
        + [pl.BlockSpec((2, BN, FA), lambda i: (0, i, 0))],
        out_shape=[out, out, out,
                   jax.ShapeDtypeStruct((2, N, FA), jnp.float32)],
    )(agg_a, agg_b, den_t, W1, W2, Wm, bm2, Wv, bv2, eps)


def _dense2_body(agg_ref, den_ref, w2_ref, w1_ref, mu_ref):
    p = jnp.concatenate([agg_ref[0], agg_ref[1]], axis=1)
    den = jnp.sum(den_ref[...], axis=0)[:, None] + 1e-16
    aggn = p / den
    dn = (((1,), (1,)), ((), ()))
    pre = lax.dot_general(aggn, w2_ref[...], dn,
                          preferred_element_type=jnp.float32, precision=_HI)
    h3 = jnp.where(pre > 0, pre, jnp.exp(jnp.minimum(pre, 0.0)) - 1.0)
    recon = lax.dot_general(h3, w1_ref[...], dn,
                            preferred_element_type=jnp.float32, precision=_HI)
    mu_ref[...] = jnp.maximum(recon, 0.0) + jnp.log(1.0 + jnp.exp(-jnp.abs(recon)))


def _dense2(agg_z, den_t, W2, W1):
    grid = (N // BN,)
    full = lambda shape: pl.BlockSpec(shape, lambda i: tuple(0 for _ in shape))
    return pl.pallas_call(
        _dense2_body,
        grid=grid,
        in_specs=[
            pl.BlockSpec((NC, BN, FA), lambda i: (0, i, 0)),
            pl.BlockSpec((NS, BN), lambda i: (i, 0)),
            full((HID, LAT)),
            full((IN_DIM, HID)),
        ],
        out_specs=[pl.BlockSpec((BN, IN_DIM), lambda i: (i, 0))],
        out_shape=[jax.ShapeDtypeStruct((N, IN_DIM), jnp.float32)],
    )(agg_z, den_t, W2, W1)[0]


def _sc_agg(table, a_src, a_dst, src_r, dst_r, want_den):
    fa = FA
    out_type = [jax.ShapeDtypeStruct((NC, N, fa), jnp.float32)]
    if want_den:
        out_type.append(
            jax.ShapeDtypeStruct((N // BN, NS, BN), jnp.float32))
    scratch = (
        [pltpu.VMEM((N,), jnp.float32),
         pltpu.VMEM((N,), jnp.float32),
         pltpu.VMEM((N,), jnp.float32),
         pltpu.VMEM((CHS, K), jnp.int32),
         pltpu.VMEM((CHS, K), jnp.int32),
         pltpu.VMEM((EPS,), jnp.float32),
         pltpu.VMEM((ZROWS, fa), jnp.float32)]
        + [pltpu.VMEM((K, fa), jnp.float32) for _ in range(NBUF)]
        + [pltpu.SemaphoreType.DMA for _ in range(NBUF + 1)]
        + [pltpu.VMEM_SHARED((N, fa), jnp.float32)]
    )

    def body(refs):
        if want_den:
            (t_hbm, a_src_hbm, a_dst_hbm, src_hbm, dst_hbm, agg_out, den_out,
             a_src_v, a_dst_v, den_v, sidx, didx, ex_v, zbuf,
             r0, r1, g0, g1, ssem, agg_sp) = refs
        else:
            (t_hbm, a_src_hbm, a_dst_hbm, src_hbm, dst_hbm, agg_out,
             a_src_v, a_dst_v, den_v, sidx, didx, ex_v, zbuf,
             r0, r1, g0, g1, ssem, agg_sp) = refs
            den_out = None
        rows = (r0, r1)
        gsem = (g0, g1)
        c = lax.axis_index("c")
        s = lax.axis_index("s")
        pltpu.sync_copy(a_src_hbm, a_src_v)
        pltpu.sync_copy(a_dst_hbm, a_dst_v)
        pltpu.sync_copy(src_hbm.at[s], sidx)
        pltpu.sync_copy(dst_hbm.at[s], didx)

        zv = jnp.zeros((LANES,), jnp.float32)

        if want_den:
            @pl.loop(0, N // LANES)
            def _(i):
                den_v[pl.ds(i * LANES, LANES)] = zv

        @pl.loop(0, ZROWS)
        def _(i):
            for f in range(fa // LANES):
                zbuf[i, pl.ds(f * LANES, LANES)] = zv

        rows_per_sub = N // NS
        for t in range(rows_per_sub // ZROWS):
            base = s * rows_per_sub + t * ZROWS
            pltpu.sync_copy(zbuf, agg_sp.at[pl.ds(base, ZROWS)])

        coff = c * N

        @pl.loop(0, CHS)
        def _(j):
            for v in range(VEC):
                si = sidx[j, pl.ds(v * LANES, LANES)]
                di = didx[j, pl.ds(v * LANES, LANES)]
                e = plsc.load_gather(a_src_v, [si]) + plsc.load_gather(a_dst_v, [di])
                e = jnp.maximum(e, NEG_SLOPE * e)
                exv = jnp.exp(e)
                ex_v[pl.ds(j * K + v * LANES, LANES)] = exv
                if want_den:
                    plsc.addupdate_scatter(den_v, [di], exv)
                sidx[j, pl.ds(v * LANES, LANES)] = si + coff

        plsc.subcore_barrier()

        def mult(j, buf):
            for v in range(VEC):
                for l in range(LANES):
                    bvec = plsc.load_gather(
                        ex_v, [jnp.full((LANES,), j * K + v * LANES + l, jnp.int32)])
                    r = v * LANES + l
                    for f in range(fa // LANES):
                        sl = (r, pl.ds(f * LANES, LANES))
                        buf[sl] = buf[sl] * bvec

        @pl.loop(0, CHS // NBUF)
        def _(h):
            j0 = NBUF * h
            gds = [pltpu.async_copy(t_hbm.at[sidx.at[j0 + b]], rows[b], gsem[b])
                   for b in range(NBUF)]
            sds = []
            for b in range(NBUF):
                gds[b].wait()
                mult(j0 + b, rows[b])
                sds.append(pltpu.async_copy(
                    rows[b], agg_sp.at[didx.at[j0 + b]], ssem, add=True))
            for d in sds:
                d.wait()

        plsc.subcore_barrier()
        rsl = pl.ds(s * rows_per_sub, rows_per_sub)
        pltpu.sync_copy(agg_sp.at[rsl], agg_out.at[c, rsl])

        if want_den:
            @pl.when(c == 0)
            def _():
                for bi in range(N // BN):
                    pltpu.sync_copy(den_v.at[pl.ds(bi * BN, BN)],
                                    den_out.at[bi, s])

    @functools.partial(pl.kernel, out_type=tuple(out_type), mesh=_mesh(),
                       scratch_types=scratch, compiler_params=_SC_PARAMS)
    def k(*refs):
        body(refs)

    return k(table, a_src, a_dst, src_r, dst_r)


def kernel(x, edge_index, W1, att_src1, att_dst1, W2, Wm, bm, Wv, bv, log_theta):
    src_r = edge_index[0].reshape(NS, CHS, K)
    dst_r = edge_index[1].reshape(NS, CHS, K)
    att2 = jnp.stack([att_src1, att_dst1], axis=1)

    a2, xq = _attn_scalars(x, W1, att2)
    a_src = a2[:, 0]
    a_dst = a2[:, 1]

    agg_a, den = _sc_agg(xq[0:2].reshape(2 * N, FA), a_src, a_dst,
                         src_r, dst_r, True)
    agg_b = _sc_agg(xq[2:4].reshape(2 * N, FA), a_src, a_dst,
                    src_r, dst_r, False)[0]
    den_t = den.reshape(N // BN * NS, BN)

    eps = jax.random.normal(jax.random.key(42), (N, LAT), jnp.float32)
    mean, log_var, z, zq = _dense1(agg_a, agg_b, den_t, W1, W2, Wm,
                                   bm.reshape(1, LAT), Wv,
                                   bv.reshape(1, LAT), eps)

    agg_z = _sc_agg(zq.reshape(2 * N, FA), a_src, a_dst, src_r, dst_r,
                    False)[0]
    mu = _dense2(agg_z, den_t, W2, W1)
    theta = jnp.exp(log_theta)
    return (mean, log_var, mu, theta, z)


# speedup vs baseline: 1.4686x; 1.1148x over previous
import functools

import jax
import jax.numpy as jnp
from jax import lax
from jax.experimental import pallas as pl
from jax.experimental.pallas import tpu as pltpu
from jax.experimental.pallas import tpu_sc as plsc

N = 10000
E = 320000
IN_DIM = 128
HID = 512
LAT = 64
NEG_SLOPE = 0.2

NC = 2
NS = 16
LANES = 16
K = 80

EPS = E // NS
CHS = EPS // K
VEC = K // LANES
FA = 32
NBUF = 2
ZROWS = 125

_HI = lax.Precision.DEFAULT
_SC_PARAMS = pltpu.CompilerParams(needs_layout_passes=False,
                                  use_tc_tiling_on_sc=False)


def _mesh():
    return plsc.VectorSubcoreMesh(
        core_axis_name="c", subcore_axis_name="s", num_cores=NC, num_subcores=NS
    )


def _attn_body(x_ref, w1_ref, att2_ref, out_ref, xq_ref):
    w12 = jnp.dot(w1_ref[...], att2_ref[...],
                  preferred_element_type=jnp.float32, precision=_HI)
    x = x_ref[...]
    out_ref[...] = jnp.dot(x, w12,
                           preferred_element_type=jnp.float32, precision=_HI)
    xq_ref[...] = jnp.stack([x[:, 0 * FA:1 * FA], x[:, 1 * FA:2 * FA],
                             x[:, 2 * FA:3 * FA], x[:, 3 * FA:4 * FA]])


def _attn_scalars(x, W1, att2):
    return pl.pallas_call(
        _attn_body,
        out_shape=[jax.ShapeDtypeStruct((N, 2), jnp.float32),
                   jax.ShapeDtypeStruct((4, N, FA), jnp.float32)],
    )(x, W1, att2)


BN = 1000


def _dense1_body(agg_a_ref, agg_b_ref, den_ref, w1_ref, w2_ref, wm_ref,
                 bm_ref, wv_ref, bv_ref, eps_ref, mean_ref, lv_ref, z_ref,
                 zq_ref):
    p = jnp.concatenate([agg_a_ref[0], agg_a_ref[1],
                         agg_b_ref[0], agg_b_ref[1]], axis=1)
    den = jnp.sum(den_ref[...], axis=0)[:, None] + 1e-16
    aggn = p / den
    out1 = jnp.dot(aggn, w1_ref[...],
                   preferred_element_type=jnp.float32, precision=_HI)
    h1 = jnp.where(out1 > 0, out1, jnp.exp(jnp.minimum(out1, 0.0)) - 1.0)
    hidden = jnp.dot(h1, w2_ref[...],
                     preferred_element_type=jnp.float32, precision=_HI)
    dn = (((1,), (1,)), ((), ()))
    mean = lax.dot_general(hidden, wm_ref[...], dn,
                           preferred_element_type=jnp.float32,
                           precision=_HI) + bm_ref[...]
    lv = lax.dot_general(hidden, wv_ref[...], dn,
                         preferred_element_type=jnp.float32,
                         precision=_HI) + bv_ref[...]
    lv = jnp.clip(lv, -10.0, 10.0)
    std = jnp.sqrt(jnp.exp(0.5 * lv) + 1e-8)
    mean_ref[...] = mean
    lv_ref[...] = lv
    z = mean + eps_ref[...] * std
    z_ref[...] = z
    zq_ref[...] = jnp.stack([z[:, :FA], z[:, FA:]])


def _dense1(agg_a, agg_b, den_t, W1, W2, Wm, bm2, Wv, bv2, eps):
    grid = (N // BN,)
    full = lambda shape: pl.BlockSpec(shape, lambda i: tuple(0 for _ in shape))
    out = jax.ShapeDtypeStruct((N, LAT), jnp.float32)
    return pl.pallas_call(
        _dense1_body,
        grid=grid,
        in_specs=[
            pl.BlockSpec((NC, BN, FA), lambda i: (0, i, 0)),
            pl.BlockSpec((NC, BN, FA), lambda i: (0, i, 0)),
            pl.BlockSpec((NS, BN), lambda i: (i, 0)),
            full((IN_DIM, HID)),
            full((HID, LAT)),
            full((LAT, LAT)),
            full((1, LAT)),
            full((LAT, LAT)),
            full((1, LAT)),
            pl.BlockSpec((BN, LAT), lambda i: (i, 0)),
        ],
        out_specs=[pl.BlockSpec((BN, LAT), lambda i: (i, 0))] * 3
        + [pl.BlockSpec((2, BN, FA), lambda i: (0, i, 0))],
        out_shape=[out, out, out,
                   jax.ShapeDtypeStruct((2, N, FA), jnp.float32)],
    )(agg_a, agg_b, den_t, W1, W2, Wm, bm2, Wv, bv2, eps)


def _dense2_body(agg_ref, den_ref, w2_ref, w1_ref, mu_ref):
    p = jnp.concatenate([agg_ref[0], agg_ref[1]], axis=1)
    den = jnp.sum(den_ref[...], axis=0)[:, None] + 1e-16
    aggn = p / den
    dn = (((1,), (1,)), ((), ()))
    pre = lax.dot_general(aggn, w2_ref[...], dn,
                          preferred_element_type=jnp.float32, precision=_HI)
    h3 = jnp.where(pre > 0, pre, jnp.exp(jnp.minimum(pre, 0.0)) - 1.0)
    recon = lax.dot_general(h3, w1_ref[...], dn,
                            preferred_element_type=jnp.float32, precision=_HI)
    mu_ref[...] = jnp.maximum(recon, 0.0) + jnp.log(1.0 + jnp.exp(-jnp.abs(recon)))


def _dense2(agg_z, den_t, W2, W1):
    grid = (N // BN,)
    full = lambda shape: pl.BlockSpec(shape, lambda i: tuple(0 for _ in shape))
    return pl.pallas_call(
        _dense2_body,
        grid=grid,
        in_specs=[
            pl.BlockSpec((NC, BN, FA), lambda i: (0, i, 0)),
            pl.BlockSpec((NS, BN), lambda i: (i, 0)),
            full((HID, LAT)),
            full((IN_DIM, HID)),
        ],
        out_specs=[pl.BlockSpec((BN, IN_DIM), lambda i: (i, 0))],
        out_shape=[jax.ShapeDtypeStruct((N, IN_DIM), jnp.float32)],
    )(agg_z, den_t, W2, W1)[0]


def _sc_agg(table, a_src, a_dst, src_r, dst_r, want_den):
    fa = FA
    out_type = [jax.ShapeDtypeStruct((NC, N, fa), jnp.float32)]
    if want_den:
        out_type.append(
            jax.ShapeDtypeStruct((N // BN, NS, BN), jnp.float32))
    scratch = (
        [pltpu.VMEM((N,), jnp.float32),
         pltpu.VMEM((N,), jnp.float32),
         pltpu.VMEM((N,), jnp.float32),
         pltpu.VMEM((CHS, K), jnp.int32),
         pltpu.VMEM((CHS, K), jnp.int32),
         pltpu.VMEM((EPS,), jnp.float32),
         pltpu.VMEM((ZROWS, fa), jnp.float32)]
        + [pltpu.VMEM((K, fa), jnp.float32) for _ in range(NBUF)]
        + [pltpu.SemaphoreType.DMA for _ in range(NBUF + 1)]
        + [pltpu.VMEM_SHARED((N, fa), jnp.float32)]
    )

    def body(refs):
        if want_den:
            (t_hbm, a_src_hbm, a_dst_hbm, src_hbm, dst_hbm, agg_out, den_out,
             a_src_v, a_dst_v, den_v, sidx, didx, ex_v, zbuf,
             r0, r1, g0, g1, ssem, agg_sp) = refs
        else:
            (t_hbm, a_src_hbm, a_dst_hbm, src_hbm, dst_hbm, agg_out,
             a_src_v, a_dst_v, den_v, sidx, didx, ex_v, zbuf,
             r0, r1, g0, g1, ssem, agg_sp) = refs
            den_out = None
        rows = (r0, r1)
        gsem = (g0, g1)
        c = lax.axis_index("c")
        s = lax.axis_index("s")
        pltpu.sync_copy(a_src_hbm, a_src_v)
        pltpu.sync_copy(a_dst_hbm, a_dst_v)
        pltpu.sync_copy(src_hbm.at[s], sidx)
        pltpu.sync_copy(dst_hbm.at[s], didx)

        zv = jnp.zeros((LANES,), jnp.float32)

        if want_den:
            @pl.loop(0, N // LANES)
            def _(i):
                den_v[pl.ds(i * LANES, LANES)] = zv

        @pl.loop(0, ZROWS)
        def _(i):
            for f in range(fa // LANES):
                zbuf[i, pl.ds(f * LANES, LANES)] = zv

        rows_per_sub = N // NS
        for t in range(rows_per_sub // ZROWS):
            base = s * rows_per_sub + t * ZROWS
            pltpu.sync_copy(zbuf, agg_sp.at[pl.ds(base, ZROWS)])

        coff = c * N

        @pl.loop(0, CHS)
        def _(j):
            for v in range(VEC):
                si = sidx[j, pl.ds(v * LANES, LANES)]
                di = didx[j, pl.ds(v * LANES, LANES)]
                e = plsc.load_gather(a_src_v, [si]) + plsc.load_gather(a_dst_v, [di])
                e = jnp.maximum(e, NEG_SLOPE * e)
                exv = jnp.exp(e)
                ex_v[pl.ds(j * K + v * LANES, LANES)] = exv
                if want_den:
                    plsc.addupdate_scatter(den_v, [di], exv)
                sidx[j, pl.ds(v * LANES, LANES)] = si + coff

        plsc.subcore_barrier()

        def mult(j, buf):
            for v in range(VEC):
                for l in range(LANES):
                    bvec = plsc.load_gather(
                        ex_v, [jnp.full((LANES,), j * K + v * LANES + l, jnp.int32)])
                    r = v * LANES + l
                    for f in range(fa // LANES):
                        sl = (r, pl.ds(f * LANES, LANES))
                        buf[sl] = buf[sl] * bvec

        @pl.loop(0, CHS // NBUF)
        def _(h):
            j0 = NBUF * h
            gds = [pltpu.async_copy(t_hbm.at[sidx.at[j0 + b]], rows[b], gsem[b])
                   for b in range(NBUF)]
            sds = []
            for b in range(NBUF):
                gds[b].wait()
                mult(j0 + b, rows[b])
                sds.append(pltpu.async_copy(
                    rows[b], agg_sp.at[didx.at[j0 + b]], ssem, add=True))
            for d in sds:
                d.wait()

        plsc.subcore_barrier()
        rsl = pl.ds(s * rows_per_sub, rows_per_sub)
        pltpu.sync_copy(agg_sp.at[rsl], agg_out.at[c, rsl])

        if want_den:
            @pl.when(c == 0)
            def _():
                for bi in range(N // BN):
                    pltpu.sync_copy(den_v.at[pl.ds(bi * BN, BN)],
                                    den_out.at[bi, s])

    @functools.partial(pl.kernel, out_type=tuple(out_type), mesh=_mesh(),
                       scratch_types=scratch, compiler_params=_SC_PARAMS)
    def k(*refs):
        body(refs)

    return k(table, a_src, a_dst, src_r, dst_r)


def kernel(x, edge_index, W1, att_src1, att_dst1, W2, Wm, bm, Wv, bv, log_theta):
    src_r = edge_index[0].reshape(NS, CHS, K)
    dst_r = edge_index[1].reshape(NS, CHS, K)
    att2 = jnp.stack([att_src1, att_dst1], axis=1)

    a2, xq = _attn_scalars(x, W1, att2)
    a_src = a2[:, 0]
    a_dst = a2[:, 1]

    agg_a, den = _sc_agg(xq[0:2].reshape(2 * N, FA), a_src, a_dst,
                         src_r, dst_r, True)
    agg_b = _sc_agg(xq[2:4].reshape(2 * N, FA), a_src, a_dst,
                    src_r, dst_r, False)[0]
    den_t = den.reshape(N // BN * NS, BN)

    eps = jax.random.normal(jax.random.key(42), (N, LAT), jnp.float32)
    mean, log_var, z, zq = _dense1(agg_a, agg_b, den_t, W1, W2, Wm,
                                   bm.reshape(1, LAT), Wv,
                                   bv.reshape(1, LAT), eps)

    agg_z = _sc_agg(zq.reshape(2 * N, FA), a_src, a_dst, src_r, dst_r,
                    False)[0]
    mu = _dense2(agg_z, den_t, W2, W1)
    theta = jnp.exp(log_theta)
    return (mean, log_var, mu, theta, z)


# speedup vs baseline: 1.4747x; 1.0042x over previous
import functools

import jax
import jax.numpy as jnp
from jax import lax
from jax.experimental import pallas as pl
from jax.experimental.pallas import tpu as pltpu
from jax.experimental.pallas import tpu_sc as plsc

N = 10000
E = 320000
IN_DIM = 128
HID = 512
LAT = 64
NEG_SLOPE = 0.2

NC = 2
NS = 16
LANES = 16
K = 80

EPS = E // NS
CHS = EPS // K
VEC = K // LANES
FA = 32
NBUF = 2
ZROWS = 125

_HI = lax.Precision.DEFAULT
_SC_PARAMS = pltpu.CompilerParams(needs_layout_passes=False,
                                  use_tc_tiling_on_sc=False)


def _mesh():
    return plsc.VectorSubcoreMesh(
        core_axis_name="c", subcore_axis_name="s", num_cores=NC, num_subcores=NS
    )


def _attn_body(x_ref, w1_ref, att2_ref, out_ref, xq_ref):
    w12 = jnp.dot(w1_ref[...], att2_ref[...],
                  preferred_element_type=jnp.float32, precision=_HI)
    x = x_ref[...]
    out_ref[...] = jnp.dot(x, w12,
                           preferred_element_type=jnp.float32, precision=_HI)
    xq_ref[...] = jnp.stack([x[:, 0 * FA:1 * FA], x[:, 1 * FA:2 * FA],
                             x[:, 2 * FA:3 * FA], x[:, 3 * FA:4 * FA]])


def _attn_scalars(x, W1, att2):
    return pl.pallas_call(
        _attn_body,
        out_shape=[jax.ShapeDtypeStruct((N, 2), jnp.float32),
                   jax.ShapeDtypeStruct((4, N, FA), jnp.float32)],
    )(x, W1, att2)


BN = 2000


def _dense1_body(agg_a_ref, agg_b_ref, den_ref, w1_ref, w2_ref, wm_ref,
                 bm_ref, wv_ref, bv_ref, eps_ref, mean_ref, lv_ref, z_ref,
                 zq_ref):
    p = jnp.concatenate([agg_a_ref[0], agg_a_ref[1],
                         agg_b_ref[0], agg_b_ref[1]], axis=1)
    den = jnp.sum(den_ref[...], axis=0)[:, None] + 1e-16
    aggn = p / den
    out1 = jnp.dot(aggn, w1_ref[...],
                   preferred_element_type=jnp.float32, precision=_HI)
    h1 = jnp.where(out1 > 0, out1, jnp.exp(jnp.minimum(out1, 0.0)) - 1.0)
    hidden = jnp.dot(h1, w2_ref[...],
                     preferred_element_type=jnp.float32, precision=_HI)
    dn = (((1,), (1,)), ((), ()))
    mean = lax.dot_general(hidden, wm_ref[...], dn,
                           preferred_element_type=jnp.float32,
                           precision=_HI) + bm_ref[...]
    lv = lax.dot_general(hidden, wv_ref[...], dn,
                         preferred_element_type=jnp.float32,
                         precision=_HI) + bv_ref[...]
    lv = jnp.clip(lv, -10.0, 10.0)
    std = jnp.sqrt(jnp.exp(0.5 * lv) + 1e-8)
    mean_ref[...] = mean
    lv_ref[...] = lv
    z = mean + eps_ref[...] * std
    z_ref[...] = z
    zq_ref[...] = jnp.stack([z[:, :FA], z[:, FA:]])


def _dense1(agg_a, agg_b, den_t, W1, W2, Wm, bm2, Wv, bv2, eps):
    grid = (N // BN,)
    full = lambda shape: pl.BlockSpec(shape, lambda i: tuple(0 for _ in shape))
    out = jax.ShapeDtypeStruct((N, LAT), jnp.float32)
    return pl.pallas_call(
        _dense1_body,
        grid=grid,
        in_specs=[
            pl.BlockSpec((NC, BN, FA), lambda i: (0, i, 0)),
            pl.BlockSpec((NC, BN, FA), lambda i: (0, i, 0)),
            pl.BlockSpec((NS, BN), lambda i: (i, 0)),
            full((IN_DIM, HID)),
            full((HID, LAT)),
            full((LAT, LAT)),
            full((1, LAT)),
            full((LAT, LAT)),
            full((1, LAT)),
            pl.BlockSpec((BN, LAT), lambda i: (i, 0)),
        ],
        out_specs=[pl.BlockSpec((BN, LAT), lambda i: (i, 0))] * 3
        + [pl.BlockSpec((2, BN, FA), lambda i: (0, i, 0))],
        out_shape=[out, out, out,
                   jax.ShapeDtypeStruct((2, N, FA), jnp.float32)],
    )(agg_a, agg_b, den_t, W1, W2, Wm, bm2, Wv, bv2, eps)


def _dense2_body(agg_ref, den_ref, w2_ref, w1_ref, mu_ref):
    p = jnp.concatenate([agg_ref[0], agg_ref[1]], axis=1)
    den = jnp.sum(den_ref[...], axis=0)[:, None] + 1e-16
    aggn = p / den
    dn = (((1,), (1,)), ((), ()))
    pre = lax.dot_general(aggn, w2_ref[...], dn,
                          preferred_element_type=jnp.float32, precision=_HI)
    h3 = jnp.where(pre > 0, pre, jnp.exp(jnp.minimum(pre, 0.0)) - 1.0)
    recon = lax.dot_general(h3, w1_ref[...], dn,
                            preferred_element_type=jnp.float32, precision=_HI)
    mu_ref[...] = jnp.maximum(recon, 0.0) + jnp.log(1.0 + jnp.exp(-jnp.abs(recon)))


def _dense2(agg_z, den_t, W2, W1):
    grid = (N // BN,)
    full = lambda shape: pl.BlockSpec(shape, lambda i: tuple(0 for _ in shape))
    return pl.pallas_call(
        _dense2_body,
        grid=grid,
        in_specs=[
            pl.BlockSpec((NC, BN, FA), lambda i: (0, i, 0)),
            pl.BlockSpec((NS, BN), lambda i: (i, 0)),
            full((HID, LAT)),
            full((IN_DIM, HID)),
        ],
        out_specs=[pl.BlockSpec((BN, IN_DIM), lambda i: (i, 0))],
        out_shape=[jax.ShapeDtypeStruct((N, IN_DIM), jnp.float32)],
    )(agg_z, den_t, W2, W1)[0]


def _sc_agg(table, a_src, a_dst, src_r, dst_r, want_den):
    fa = FA
    out_type = [jax.ShapeDtypeStruct((NC, N, fa), jnp.float32)]
    if want_den:
        out_type.append(
            jax.ShapeDtypeStruct((N // BN, NS, BN), jnp.float32))
    scratch = (
        [pltpu.VMEM((N,), jnp.float32),
         pltpu.VMEM((N,), jnp.float32),
         pltpu.VMEM((N,), jnp.float32),
         pltpu.VMEM((CHS, K), jnp.int32),
         pltpu.VMEM((CHS, K), jnp.int32),
         pltpu.VMEM((EPS,), jnp.float32),
         pltpu.VMEM((ZROWS, fa), jnp.float32)]
        + [pltpu.VMEM((K, fa), jnp.float32) for _ in range(NBUF)]
        + [pltpu.SemaphoreType.DMA for _ in range(NBUF + 1)]
        + [pltpu.VMEM_SHARED((N, fa), jnp.float32)]
    )

    def body(refs):
        if want_den:
            (t_hbm, a_src_hbm, a_dst_hbm, src_hbm, dst_hbm, agg_out, den_out,
             a_src_v, a_dst_v, den_v, sidx, didx, ex_v, zbuf,
             r0, r1, g0, g1, ssem, agg_sp) = refs
        else:
            (t_hbm, a_src_hbm, a_dst_hbm, src_hbm, dst_hbm, agg_out,
             a_src_v, a_dst_v, den_v, sidx, didx, ex_v, zbuf,
             r0, r1, g0, g1, ssem, agg_sp) = refs
            den_out = None
        rows = (r0, r1)
        gsem = (g0, g1)
        c = lax.axis_index("c")
        s = lax.axis_index("s")
        pltpu.sync_copy(a_src_hbm, a_src_v)
        pltpu.sync_copy(a_dst_hbm, a_dst_v)
        pltpu.sync_copy(src_hbm.at[s], sidx)
        pltpu.sync_copy(dst_hbm.at[s], didx)

        zv = jnp.zeros((LANES,), jnp.float32)

        if want_den:
            @pl.loop(0, N // LANES)
            def _(i):
                den_v[pl.ds(i * LANES, LANES)] = zv

        @pl.loop(0, ZROWS)
        def _(i):
            for f in range(fa // LANES):
                zbuf[i, pl.ds(f * LANES, LANES)] = zv

        rows_per_sub = N // NS
        for t in range(rows_per_sub // ZROWS):
            base = s * rows_per_sub + t * ZROWS
            pltpu.sync_copy(zbuf, agg_sp.at[pl.ds(base, ZROWS)])

        coff = c * N

        @pl.loop(0, CHS)
        def _(j):
            for v in range(VEC):
                si = sidx[j, pl.ds(v * LANES, LANES)]
                di = didx[j, pl.ds(v * LANES, LANES)]
                e = plsc.load_gather(a_src_v, [si]) + plsc.load_gather(a_dst_v, [di])
                e = jnp.maximum(e, NEG_SLOPE * e)
                exv = jnp.exp(e)
                ex_v[pl.ds(j * K + v * LANES, LANES)] = exv
                if want_den:
                    plsc.addupdate_scatter(den_v, [di], exv)
                sidx[j, pl.ds(v * LANES, LANES)] = si + coff

        plsc.subcore_barrier()

        def mult(j, buf):
            for v in range(VEC):
                for l in range(LANES):
                    bvec = plsc.load_gather(
                        ex_v, [jnp.full((LANES,), j * K + v * LANES + l, jnp.int32)])
                    r = v * LANES + l
                    for f in range(fa // LANES):
                        sl = (r, pl.ds(f * LANES, LANES))
                        buf[sl] = buf[sl] * bvec

        @pl.loop(0, CHS // NBUF)
        def _(h):
            j0 = NBUF * h
            gds = [pltpu.async_copy(t_hbm.at[sidx.at[j0 + b]], rows[b], gsem[b])
                   for b in range(NBUF)]
            sds = []
            for b in range(NBUF):
                gds[b].wait()
                mult(j0 + b, rows[b])
                sds.append(pltpu.async_copy(
                    rows[b], agg_sp.at[didx.at[j0 + b]], ssem, add=True))
            for d in sds:
                d.wait()

        plsc.subcore_barrier()
        rsl = pl.ds(s * rows_per_sub, rows_per_sub)
        pltpu.sync_copy(agg_sp.at[rsl], agg_out.at[c, rsl])

        if want_den:
            @pl.when(c == 0)
            def _():
                for bi in range(N // BN):
                    pltpu.sync_copy(den_v.at[pl.ds(bi * BN, BN)],
                                    den_out.at[bi, s])

    @functools.partial(pl.kernel, out_type=tuple(out_type), mesh=_mesh(),
                       scratch_types=scratch, compiler_params=_SC_PARAMS)
    def k(*refs):
        body(refs)

    return k(table, a_src, a_dst, src_r, dst_r)


def kernel(x, edge_index, W1, att_src1, att_dst1, W2, Wm, bm, Wv, bv, log_theta):
    src_r = edge_index[0].reshape(NS, CHS, K)
    dst_r = edge_index[1].reshape(NS, CHS, K)
    att2 = jnp.stack([att_src1, att_dst1], axis=1)

    a2, xq = _attn_scalars(x, W1, att2)
    a_src = a2[:, 0]
    a_dst = a2[:, 1]

    agg_a, den = _sc_agg(xq[0:2].reshape(2 * N, FA), a_src, a_dst,
                         src_r, dst_r, True)
    agg_b = _sc_agg(xq[2:4].reshape(2 * N, FA), a_src, a_dst,
                    src_r, dst_r, False)[0]
    den_t = den.reshape(N // BN * NS, BN)

    eps = jax.random.normal(jax.random.key(42), (N, LAT), jnp.float32)
    mean, log_var, z, zq = _dense1(agg_a, agg_b, den_t, W1, W2, Wm,
                                   bm.reshape(1, LAT), Wv,
                                   bv.reshape(1, LAT), eps)

    agg_z = _sc_agg(zq.reshape(2 * N, FA), a_src, a_dst, src_r, dst_r,
                    False)[0]
    mu = _dense2(agg_z, den_t, W2, W1)
    theta = jnp.exp(log_theta)
    return (mean, log_var, mu, theta, z)
